# baseline (speedup 1.0000x reference)
import functools

import jax
import jax.numpy as jnp
from jax import lax
from jax.experimental import pallas as pl
from jax.experimental.pallas import tpu as pltpu
from jax.experimental.pallas import tpu_sc as plsc

_NUM_CLASSES = 21
_OVERLAP_THRESH = 0.5
_NEG_POS_RATIO = 3
_V0, _V1 = 0.1, 0.2
_LANES = 128
_IMGS_PER_STEP = 16
_SCL = 16


def _sc_match_body(nobj, ap, anch_hbm, tgt_hbm, out_hbm, anch_v, tgt_v,
                   btv_v, bti_v, pk_v):
    wid = lax.axis_index("s") * 2 + lax.axis_index("c")
    pltpu.sync_copy(anch_hbm.at[pl.ds(wid * 4 * ap, 4 * ap)], anch_v)
    pltpu.sync_copy(tgt_hbm.at[pl.ds(wid * nobj * 8, nobj * 8)],
                    tgt_v.at[pl.ds(0, nobj * 8)])

    lane = lax.iota(jnp.int32, _SCL)
    nvec = ap // _SCL

    grow = [tgt_v[pl.ds(8 * g, _SCL)] for g in range(nobj)]
    gx1 = [grow[g][0] for g in range(nobj)]
    gy1 = [grow[g][1] for g in range(nobj)]
    gx2 = [grow[g][2] for g in range(nobj)]
    gy2 = [grow[g][3] for g in range(nobj)]
    glab = [grow[g][4] for g in range(nobj)]
    gvalid = [jnp.broadcast_to(glab[g], (_SCL,)) > 0.0 for g in range(nobj)]
    garea = [(gx2[g] - gx1[g]) * (gy2[g] - gy1[g]) for g in range(nobj)]

    zero = jnp.zeros((_SCL,), jnp.float32)
    neg1 = jnp.full((_SCL,), -1.0, jnp.float32)

    def pass1(i, carry):
        gmax = list(carry[:nobj])
        gidx = list(carry[nobj:])
        cx = anch_v[pl.ds(0 * ap + i * _SCL, _SCL)]
        cy = anch_v[pl.ds(1 * ap + i * _SCL, _SCL)]
        w = anch_v[pl.ds(2 * ap + i * _SCL, _SCL)]
        h = anch_v[pl.ds(3 * ap + i * _SCL, _SCL)]
        ax1 = cx - w / 2.0
        ay1 = cy - h / 2.0
        ax2 = cx + w / 2.0
        ay2 = cy + h / 2.0
        area_a = (ax2 - ax1) * (ay2 - ay1)
        aidx = lane + i * _SCL
        bt_val = neg1
        bt_idx = jnp.zeros((_SCL,), jnp.int32)
        for g in range(nobj):
            wx = jnp.maximum(
                jnp.minimum(gx2[g], ax2) - jnp.maximum(gx1[g], ax1), 0.0)
            wy = jnp.maximum(
                jnp.minimum(gy2[g], ay2) - jnp.maximum(gy1[g], ay1), 0.0)
            inter = wx * wy
            iou = inter / (garea[g] + area_a - inter)
            updg = iou > gmax[g]
            gmax[g] = jnp.where(updg, iou, gmax[g])
            gidx[g] = jnp.where(updg, aidx, gidx[g])
            val = jnp.where(gvalid[g], iou, neg1)
            upd = val > bt_val
            bt_val = jnp.where(upd, val, bt_val)
            bt_idx = jnp.where(upd, g, bt_idx)
        btv_v[pl.ds(i * _SCL, _SCL)] = bt_val
        bti_v[pl.ds(i * _SCL, _SCL)] = bt_idx
        return tuple(gmax) + tuple(gidx)

    init = tuple([neg1] * nobj) + tuple([jnp.zeros((_SCL,), jnp.int32)] * nobj)
    fin = lax.fori_loop(0, nvec, pass1, init, unroll=False)
    gmax = fin[:nobj]
    gidx = fin[nobj:]

    bp_eff = []
    for g in range(nobj):
        gm = jnp.max(gmax[g])
        bp = jnp.min(jnp.where(gmax[g] == gm, gidx[g], ap))
        v = (glab[g] > 0.0).astype(jnp.int32)
        bp_eff.append(bp * v + (v - 1))

    def pass2(i, carry):
        bt_val = btv_v[pl.ds(i * _SCL, _SCL)]
        bt_idx = bti_v[pl.ds(i * _SCL, _SCL)]
        aidx = lane + i * _SCL
        for g in range(nobj):
            hit = aidx == bp_eff[g]
            bt_val = jnp.where(hit, 2.0, bt_val)
            bt_idx = jnp.where(hit, g, bt_idx)
        flag = bt_val >= _OVERLAP_THRESH
        pk_v[pl.ds(i * _SCL, _SCL)] = bt_idx + jnp.where(
            flag, nobj, 0).astype(jnp.int32)
        return carry

    lax.fori_loop(0, nvec, pass2, 0, unroll=False)
    pltpu.sync_copy(pk_v, out_hbm.at[pl.ds(wid * ap, ap)])


def _sc_match(anch_flat, tgt_flat, batch, nobj, ap):
    mesh = plsc.VectorSubcoreMesh(core_axis_name="c", subcore_axis_name="s")
    body = functools.partial(_sc_match_body, nobj, ap)
    f = pl.kernel(
        body,
        mesh=mesh,
        compiler_params=pltpu.CompilerParams(needs_layout_passes=False),
        out_type=jax.ShapeDtypeStruct((batch * ap,), jnp.int32),
        scratch_types=[
            pltpu.VMEM((4 * ap,), jnp.float32),
            pltpu.VMEM((nobj * 8 + 8,), jnp.float32),
            pltpu.VMEM((ap,), jnp.float32),
            pltpu.VMEM((ap,), jnp.int32),
            pltpu.VMEM((ap,), jnp.int32),
        ],
    )
    return f(anch_flat, tgt_flat)


def _image_losses(nobj, num_anchors, sub, j, loc_ref, conf_ref, anch_ref,
                  ign_ref, tgt_ref, match_ref):
    shape = (sub, _LANES)

    packed = match_ref[j]
    bt_idx = lax.rem(packed, nobj)
    flag = packed >= nobj

    mx1 = jnp.zeros(shape, jnp.float32)
    my1 = jnp.zeros(shape, jnp.float32)
    mx2 = jnp.zeros(shape, jnp.float32)
    my2 = jnp.zeros(shape, jnp.float32)
    mlab = jnp.zeros(shape, jnp.float32)
    for g in range(nobj):
        sel = bt_idx == g
        mx1 = jnp.where(sel, tgt_ref[j, g, 0], mx1)
        my1 = jnp.where(sel, tgt_ref[j, g, 1], my1)
        mx2 = jnp.where(sel, tgt_ref[j, g, 2], mx2)
        my2 = jnp.where(sel, tgt_ref[j, g, 3], my2)
        mlab = jnp.where(sel, tgt_ref[j, g, 4], mlab)
    conf_t = jnp.where(flag, mlab.astype(jnp.int32), 0)
    pos = conf_t > 0
    posf = pos.astype(jnp.float32)

    acx = anch_ref[j, 0]
    acy = anch_ref[j, 1]
    aw = anch_ref[j, 2]
    ah = anch_ref[j, 3]
    enc = (
        ((mx1 + mx2) / 2.0 - acx) / (_V0 * aw),
        ((my1 + my2) / 2.0 - acy) / (_V0 * ah),
        jnp.log((mx2 - mx1) / aw) / _V1,
        jnp.log((my2 - my1) / ah) / _V1,
    )
    loss_l = jnp.float32(0.0)
    for c in range(4):
        ad = jnp.abs(loc_ref[j, c] - enc[c])
        sl1 = jnp.where(ad < 1.0, 0.5 * ad * ad, ad - 0.5)
        loss_l += jnp.sum(sl1 * posf)

    cp = conf_ref[j]
    cmax = jnp.max(cp, axis=0)
    s = jnp.sum(jnp.exp(cp - cmax[None]), axis=0)
    lse = cmax + jnp.log(s)
    gath = jnp.zeros(shape, jnp.float32)
    for c in range(_NUM_CLASSES):
        gath = jnp.where(conf_t == c, cp[c], gath)
    raw = lse - gath
    ignore = ign_ref[j] > 0.0
    proxy = jnp.where(jnp.logical_or(pos, ignore), 0.0, raw)

    num_pos = jnp.sum(pos.astype(jnp.int32))
    max_neg = jnp.sum((proxy > 0.0).astype(jnp.int32))
    k = jnp.minimum(_NEG_POS_RATIO * num_pos, max_neg)
    loss_c_pos = jnp.sum(jnp.where(pos, raw, 0.0))
    return loss_l, loss_c_pos, num_pos.astype(jnp.float32), proxy, k


def _topk_sums(proxies, ks):
    n = len(proxies)
    pbits = [lax.bitcast_convert_type(p, jnp.int32) for p in proxies]

    def bis(_, los_his):
        los, his = los_his
        nlos, nhis = [], []
        cs = []
        for j in range(n):
            mid = los[j] + lax.shift_right_logical(his[j] - los[j], 1)
            cs.append((mid, jnp.sum((pbits[j] > mid).astype(jnp.int32))))
        for j in range(n):
            mid, c = cs[j]
            take_hi = c < ks[j]
            nlos.append(jnp.where(take_hi, los[j], mid + 1))
            nhis.append(jnp.where(take_hi, mid, his[j]))
        return tuple(nlos), tuple(nhis)

    init = (tuple([jnp.int32(0)] * n), tuple([jnp.int32(0x7F800000)] * n))
    los, _ = lax.fori_loop(0, 31, bis, init)
    total = jnp.float32(0.0)
    for j in range(n):
        lo = los[j]
        gt_mask = pbits[j] > lo
        cnt_gt = jnp.sum(gt_mask.astype(jnp.int32))
        sum_gt = jnp.sum(jnp.where(gt_mask, proxies[j], 0.0))
        tval = jnp.max(jnp.where(pbits[j] == lo, proxies[j], 0.0))
        topk = sum_gt + (ks[j] - cnt_gt).astype(jnp.float32) * tval
        total += jnp.where(ks[j] > 0, topk, 0.0)
    return total


def _odm_body(nobj, num_anchors, nsteps, sub, loc_ref, conf_ref, anch_ref,
              ign_ref, tgt_ref, match_ref, out_ref, acc_ref):
    i = pl.program_id(0)
    ll = jnp.float32(0.0)
    lc = jnp.float32(0.0)
    npos = jnp.float32(0.0)
    proxies, ks = [], []
    for j in range(_IMGS_PER_STEP):
        a, b, c, proxy, k = _image_losses(nobj, num_anchors, sub, j, loc_ref,
                                          conf_ref, anch_ref, ign_ref,
                                          tgt_ref, match_ref)
        ll += a
        lc += b
        npos += c
        proxies.append(proxy)
        ks.append(k)
    lc += _topk_sums(proxies, ks)

    @pl.when(i == 0)
    def _():
        acc_ref[0] = ll
        acc_ref[1] = lc
        acc_ref[2] = npos

    @pl.when(i > 0)
    def _():
        acc_ref[0] += ll
        acc_ref[1] += lc
        acc_ref[2] += npos

    @pl.when(i == nsteps - 1)
    def _():
        total = acc_ref[2]
        out_ref[0] = acc_ref[0] / total
        out_ref[1] = acc_ref[1] / total


def kernel(loc_pred, conf_pred, refined_anchors, ignore_flags_refined_anchor,
           targets):
    B, A, C = conf_pred.shape
    nobj = targets.shape[1]
    ap = ((A + _LANES - 1) // _LANES) * _LANES
    sub = ap // _LANES
    padn = ap - A
    m = _IMGS_PER_STEP
    nsteps = B // m

    anch_pad = jnp.broadcast_to(
        jnp.array([-100.0, -100.0, 1e-4, 1e-4], jnp.float32), (B, padn, 4))
    anch_p = jnp.concatenate([refined_anchors, anch_pad], axis=1)
    anch_p = anch_p.transpose(0, 2, 1).reshape(B, 4, sub, _LANES)
    tgt_pad = jnp.pad(targets, ((0, 0), (0, 0), (0, 3)))

    match = _sc_match(anch_p.reshape(B * 4 * ap), tgt_pad.reshape(B * nobj * 8),
                      B, nobj, ap)
    match_p = match.reshape(B, sub, _LANES)

    conf_p = jnp.zeros((B, C, sub, _LANES), jnp.float32)
    loc_p = jnp.pad(loc_pred, ((0, 0), (0, padn), (0, 0)))
    loc_p = loc_p.transpose(0, 2, 1).reshape(B, 4, sub, _LANES)
    ign_p = jnp.pad(ignore_flags_refined_anchor, ((0, 0), (0, padn)),
                    constant_values=1.0).reshape(B, sub, _LANES)

    body = functools.partial(_odm_body, nobj, A, nsteps, sub)
    out = pl.pallas_call(
        body,
        grid=(nsteps,),
        in_specs=[
            pl.BlockSpec((m, 4, sub, _LANES), lambda i: (i, 0, 0, 0)),
            pl.BlockSpec((m, C, sub, _LANES), lambda i: (i, 0, 0, 0)),
            pl.BlockSpec((m, 4, sub, _LANES), lambda i: (i, 0, 0, 0)),
            pl.BlockSpec((m, sub, _LANES), lambda i: (i, 0, 0)),
            pl.BlockSpec((m, nobj, 5), lambda i: (i, 0, 0),
                         memory_space=pltpu.SMEM),
            pl.BlockSpec((m, sub, _LANES), lambda i: (i, 0, 0)),
        ],
        out_specs=pl.BlockSpec((2,), lambda i: (0,),
                               memory_space=pltpu.SMEM),
        out_shape=jax.ShapeDtypeStruct((2,), jnp.float32),
        scratch_shapes=[pltpu.SMEM((3,), jnp.float32)],
    )(loc_p, conf_p, anch_p, ign_p, targets, match_p)
    return out[0], out[1]


# speedup vs baseline: 1.0648x; 1.0648x over previous
import functools

import jax
import jax.numpy as jnp
from jax import lax
from jax.experimental import pallas as pl
from jax.experimental.pallas import tpu as pltpu
from jax.experimental.pallas import tpu_sc as plsc

_NUM_CLASSES = 21
_OVERLAP_THRESH = 0.5
_NEG_POS_RATIO = 3
_V0, _V1 = 0.1, 0.2
_LANES = 128
_IMGS_PER_STEP = 16
_SCL = 16


def _sc_match_body(nobj, ap, anch_hbm, tgt_hbm, out_hbm, anch_v, tgt_v,
                   btv_v, bti_v, pk_v):
    wid = lax.axis_index("s") * 2 + lax.axis_index("c")
    pltpu.sync_copy(anch_hbm.at[pl.ds(wid * 4 * ap, 4 * ap)], anch_v)
    pltpu.sync_copy(tgt_hbm.at[pl.ds(wid * nobj * 8, nobj * 8)],
                    tgt_v.at[pl.ds(0, nobj * 8)])

    lane = lax.iota(jnp.int32, _SCL)
    nvec = ap // _SCL

    grow = [tgt_v[pl.ds(8 * g, _SCL)] for g in range(nobj)]
    gx1 = [grow[g][0] for g in range(nobj)]
    gy1 = [grow[g][1] for g in range(nobj)]
    gx2 = [grow[g][2] for g in range(nobj)]
    gy2 = [grow[g][3] for g in range(nobj)]
    glab = [grow[g][4] for g in range(nobj)]
    gvalid = [jnp.broadcast_to(glab[g], (_SCL,)) > 0.0 for g in range(nobj)]
    garea = [(gx2[g] - gx1[g]) * (gy2[g] - gy1[g]) for g in range(nobj)]

    zero = jnp.zeros((_SCL,), jnp.float32)
    neg1 = jnp.full((_SCL,), -1.0, jnp.float32)

    def pass1(i, carry):
        gmax = list(carry[:nobj])
        gidx = list(carry[nobj:])
        cx = anch_v[pl.ds(0 * ap + i * _SCL, _SCL)]
        cy = anch_v[pl.ds(1 * ap + i * _SCL, _SCL)]
        w = anch_v[pl.ds(2 * ap + i * _SCL, _SCL)]
        h = anch_v[pl.ds(3 * ap + i * _SCL, _SCL)]
        ax1 = cx - w / 2.0
        ay1 = cy - h / 2.0
        ax2 = cx + w / 2.0
        ay2 = cy + h / 2.0
        area_a = (ax2 - ax1) * (ay2 - ay1)
        aidx = lane + i * _SCL
        bt_val = neg1
        bt_idx = jnp.zeros((_SCL,), jnp.int32)
        for g in range(nobj):
            wx = jnp.maximum(
                jnp.minimum(gx2[g], ax2) - jnp.maximum(gx1[g], ax1), 0.0)
            wy = jnp.maximum(
                jnp.minimum(gy2[g], ay2) - jnp.maximum(gy1[g], ay1), 0.0)
            inter = wx * wy
            iou = inter / (garea[g] + area_a - inter)
            updg = iou > gmax[g]
            gmax[g] = jnp.where(updg, iou, gmax[g])
            gidx[g] = jnp.where(updg, aidx, gidx[g])
            val = jnp.where(gvalid[g], iou, neg1)
            upd = val > bt_val
            bt_val = jnp.where(upd, val, bt_val)
            bt_idx = jnp.where(upd, g, bt_idx)
        btv_v[pl.ds(i * _SCL, _SCL)] = bt_val
        bti_v[pl.ds(i * _SCL, _SCL)] = bt_idx
        return tuple(gmax) + tuple(gidx)

    init = tuple([neg1] * nobj) + tuple([jnp.zeros((_SCL,), jnp.int32)] * nobj)
    fin = lax.fori_loop(0, nvec, pass1, init, unroll=False)
    gmax = fin[:nobj]
    gidx = fin[nobj:]

    bp_eff = []
    for g in range(nobj):
        gm = jnp.max(gmax[g])
        bp = jnp.min(jnp.where(gmax[g] == gm, gidx[g], ap))
        v = (glab[g] > 0.0).astype(jnp.int32)
        bp_eff.append(bp * v + (v - 1))

    def pass2(i, carry):
        bt_val = btv_v[pl.ds(i * _SCL, _SCL)]
        bt_idx = bti_v[pl.ds(i * _SCL, _SCL)]
        aidx = lane + i * _SCL
        for g in range(nobj):
            hit = aidx == bp_eff[g]
            bt_val = jnp.where(hit, 2.0, bt_val)
            bt_idx = jnp.where(hit, g, bt_idx)
        flag = bt_val >= _OVERLAP_THRESH
        pk_v[pl.ds(i * _SCL, _SCL)] = bt_idx + jnp.where(
            flag, nobj, 0).astype(jnp.int32)
        return carry

    lax.fori_loop(0, nvec, pass2, 0, unroll=False)
    pltpu.sync_copy(pk_v, out_hbm.at[pl.ds(wid * ap, ap)])


def _sc_match(anch_flat, tgt_flat, batch, nobj, ap):
    mesh = plsc.VectorSubcoreMesh(core_axis_name="c", subcore_axis_name="s")
    body = functools.partial(_sc_match_body, nobj, ap)
    f = pl.kernel(
        body,
        mesh=mesh,
        compiler_params=pltpu.CompilerParams(needs_layout_passes=False),
        out_type=jax.ShapeDtypeStruct((batch * ap,), jnp.int32),
        scratch_types=[
            pltpu.VMEM((4 * ap,), jnp.float32),
            pltpu.VMEM((nobj * 8 + 8,), jnp.float32),
            pltpu.VMEM((ap,), jnp.float32),
            pltpu.VMEM((ap,), jnp.int32),
            pltpu.VMEM((ap,), jnp.int32),
        ],
    )
    return f(anch_flat, tgt_flat)


def _image_losses(nobj, num_anchors, sub, j, loc_ref, conf_ref, anch_ref,
                  ign_ref, tgt_ref, match_ref):
    shape = (sub, _LANES)

    packed = match_ref[j]
    bt_idx = lax.rem(packed, nobj)
    flag = packed >= nobj

    mx1 = jnp.zeros(shape, jnp.float32)
    my1 = jnp.zeros(shape, jnp.float32)
    mx2 = jnp.zeros(shape, jnp.float32)
    my2 = jnp.zeros(shape, jnp.float32)
    mlab = jnp.zeros(shape, jnp.float32)
    for g in range(nobj):
        sel = bt_idx == g
        mx1 = jnp.where(sel, tgt_ref[j, g, 0], mx1)
        my1 = jnp.where(sel, tgt_ref[j, g, 1], my1)
        mx2 = jnp.where(sel, tgt_ref[j, g, 2], mx2)
        my2 = jnp.where(sel, tgt_ref[j, g, 3], my2)
        mlab = jnp.where(sel, tgt_ref[j, g, 4], mlab)
    conf_t = jnp.where(flag, mlab.astype(jnp.int32), 0)
    pos = conf_t > 0
    posf = pos.astype(jnp.float32)

    acx = anch_ref[j, 0]
    acy = anch_ref[j, 1]
    aw = anch_ref[j, 2]
    ah = anch_ref[j, 3]
    enc = (
        ((mx1 + mx2) / 2.0 - acx) / (_V0 * aw),
        ((my1 + my2) / 2.0 - acy) / (_V0 * ah),
        jnp.log((mx2 - mx1) / aw) / _V1,
        jnp.log((my2 - my1) / ah) / _V1,
    )
    loss_l = jnp.float32(0.0)
    for c in range(4):
        ad = jnp.abs(loc_ref[j, c] - enc[c])
        sl1 = jnp.where(ad < 1.0, 0.5 * ad * ad, ad - 0.5)
        loss_l += jnp.sum(sl1 * posf)

    cp = conf_ref[j]
    cmax = jnp.max(cp, axis=0)
    s = jnp.sum(jnp.exp(cp - cmax[None]), axis=0)
    lse = cmax + jnp.log(s)
    gath = jnp.zeros(shape, jnp.float32)
    for c in range(_NUM_CLASSES):
        gath = jnp.where(conf_t == c, cp[c], gath)
    raw = lse - gath
    ignore = ign_ref[j] > 0.0
    proxy = jnp.where(jnp.logical_or(pos, ignore), 0.0, raw)

    num_pos = jnp.sum(pos.astype(jnp.int32))
    max_neg = jnp.sum((proxy > 0.0).astype(jnp.int32))
    k = jnp.minimum(_NEG_POS_RATIO * num_pos, max_neg)
    loss_c_pos = jnp.sum(jnp.where(pos, raw, 0.0))
    return loss_l, loss_c_pos, num_pos.astype(jnp.float32), proxy, k


def _topk_sums(proxies, ks):
    n = len(proxies)
    pbits = [lax.bitcast_convert_type(p, jnp.int32) for p in proxies]

    def bis(_, los_his):
        los, his = los_his
        nlos, nhis = [], []
        cs = []
        for j in range(n):
            mid = los[j] + lax.shift_right_logical(his[j] - los[j], 1)
            cs.append((mid, jnp.sum((pbits[j] > mid).astype(jnp.int32))))
        for j in range(n):
            mid, c = cs[j]
            take_hi = c < ks[j]
            nlos.append(jnp.where(take_hi, los[j], mid + 1))
            nhis.append(jnp.where(take_hi, mid, his[j]))
        return tuple(nlos), tuple(nhis)

    init = (tuple([jnp.int32(0)] * n), tuple([jnp.int32(0x7F800000)] * n))
    los, _ = lax.fori_loop(0, 31, bis, init)
    total = jnp.float32(0.0)
    for j in range(n):
        lo = los[j]
        gt_mask = pbits[j] > lo
        cnt_gt = jnp.sum(gt_mask.astype(jnp.int32))
        sum_gt = jnp.sum(jnp.where(gt_mask, proxies[j], 0.0))
        tval = jnp.max(jnp.where(pbits[j] == lo, proxies[j], 0.0))
        topk = sum_gt + (ks[j] - cnt_gt).astype(jnp.float32) * tval
        total += jnp.where(ks[j] > 0, topk, 0.0)
    return total


def _odm_body(nobj, num_anchors, nsteps, sub, loc_ref, conf_ref, anch_ref,
              ign_ref, tgt_ref, match_ref, out_ref, acc_ref):
    i = pl.program_id(0)
    ll = jnp.float32(0.0)
    lc = jnp.float32(0.0)
    npos = jnp.float32(0.0)
    proxies, ks = [], []
    for j in range(_IMGS_PER_STEP):
        a, b, c, proxy, k = _image_losses(nobj, num_anchors, sub, j, loc_ref,
                                          conf_ref, anch_ref, ign_ref,
                                          tgt_ref, match_ref)
        ll += a
        lc += b
        npos += c
        proxies.append(proxy)
        ks.append(k)
    lc += _topk_sums(proxies, ks)

    @pl.when(i == 0)
    def _():
        acc_ref[0] = ll
        acc_ref[1] = lc
        acc_ref[2] = npos

    @pl.when(i > 0)
    def _():
        acc_ref[0] += ll
        acc_ref[1] += lc
        acc_ref[2] += npos

    @pl.when(i == nsteps - 1)
    def _():
        total = acc_ref[2]
        out_ref[0] = acc_ref[0] / total
        out_ref[1] = acc_ref[1] / total


def kernel(loc_pred, conf_pred, refined_anchors, ignore_flags_refined_anchor,
           targets):
    B, A, C = conf_pred.shape
    nobj = targets.shape[1]
    ap = ((A + _LANES - 1) // _LANES) * _LANES
    sub = ap // _LANES
    padn = ap - A
    m = _IMGS_PER_STEP
    nsteps = B // m

    anch_pad = jnp.broadcast_to(
        jnp.array([-100.0, -100.0, 1e-4, 1e-4], jnp.float32), (B, padn, 4))
    anch_p = jnp.concatenate([refined_anchors, anch_pad], axis=1)
    anch_p = anch_p.transpose(0, 2, 1).reshape(B, 4, sub, _LANES)
    tgt_pad = jnp.pad(targets, ((0, 0), (0, 0), (0, 3)))

    match = _sc_match(anch_p.reshape(B * 4 * ap), tgt_pad.reshape(B * nobj * 8),
                      B, nobj, ap)
    match_p = match.reshape(B, sub, _LANES)

    conf_p = jnp.pad(conf_pred, ((0, 0), (0, padn), (0, 0)))
    conf_p = conf_p.transpose(0, 2, 1).reshape(B, C, sub, _LANES)
    loc_p = jnp.pad(loc_pred, ((0, 0), (0, padn), (0, 0)))
    loc_p = loc_p.transpose(0, 2, 1).reshape(B, 4, sub, _LANES)
    ign_p = jnp.pad(ignore_flags_refined_anchor, ((0, 0), (0, padn)),
                    constant_values=1.0).reshape(B, sub, _LANES)

    body = functools.partial(_odm_body, nobj, A, nsteps, sub)
    out = pl.pallas_call(
        body,
        grid=(nsteps,),
        in_specs=[
            pl.BlockSpec((m, 4, sub, _LANES), lambda i: (i, 0, 0, 0)),
            pl.BlockSpec((m, C, sub, _LANES), lambda i: (i, 0, 0, 0)),
            pl.BlockSpec((m, 4, sub, _LANES), lambda i: (i, 0, 0, 0)),
            pl.BlockSpec((m, sub, _LANES), lambda i: (i, 0, 0)),
            pl.BlockSpec((m, nobj, 5), lambda i: (i, 0, 0),
                         memory_space=pltpu.SMEM),
            pl.BlockSpec((m, sub, _LANES), lambda i: (i, 0, 0)),
        ],
        out_specs=pl.BlockSpec((2,), lambda i: (0,),
                               memory_space=pltpu.SMEM),
        out_shape=jax.ShapeDtypeStruct((2,), jnp.float32),
        scratch_shapes=[pltpu.SMEM((3,), jnp.float32)],
    )(loc_p, conf_p, anch_p, ign_p, targets, match_p)
    return out[0], out[1]
